# baseline (speedup 1.0000x reference)
import functools

import jax
import jax.numpy as jnp
from jax.experimental import pallas as pl
from jax.experimental.pallas import tpu as pltpu
from jax.experimental.pallas import tpu_sc as plsc

_SEQ = 512
_B = 8
_ROWS = _SEQ * _B
_NC = 2
_NS = 16
_NW = _NC * _NS
_RPW = _ROWS // _NW
_L = 16
_S_IN = 64


def _sc_mesh():
    return plsc.VectorSubcoreMesh(core_axis_name="c", subcore_axis_name="s")


def _worker_id():
    return jax.lax.axis_index("s") * _NC + jax.lax.axis_index("c")


def _sc_scatter(ids_flat, nums_flat, zeros2d):
    din = 512

    @functools.partial(
        pl.kernel,
        mesh=_sc_mesh(),
        compiler_params=pltpu.CompilerParams(
            needs_layout_passes=False, use_tc_tiling_on_sc=True),
        out_type=jax.ShapeDtypeStruct((_ROWS, din), jnp.float32),
        scratch_types=[
            pltpu.VMEM((_RPW * _S_IN,), jnp.int32),
            pltpu.VMEM((_RPW,), jnp.int32),
            pltpu.VMEM((_RPW, din), jnp.float32),
        ],
    )
    def k(ids_hbm, nums_hbm, z_hbm, out_hbm, ids_v, nums_v, buf_v):
        wid = _worker_id()
        base = wid * _RPW
        pltpu.sync_copy(ids_hbm.at[pl.ds(base * _S_IN, _RPW * _S_IN)], ids_v)
        pltpu.sync_copy(nums_hbm.at[pl.ds(base, _RPW)], nums_v)
        pltpu.sync_copy(z_hbm, buf_v)
        ones = jnp.full((_L,), 1.0, jnp.float32)
        lane = jax.lax.iota(jnp.int32, _L)

        def group(g, carry):
            numv = nums_v[pl.ds(g * _L, _L)]
            for j in range(_L):
                r = g * _L + j
                num = numv[j]
                rsp = jnp.full((_L,), 0, jnp.int32) + r
                for c in range(_S_IN // _L):
                    idv = ids_v[pl.ds(r * _S_IN + c * _L, _L)]
                    m = (lane + (c * _L)) < num
                    plsc.store_scatter(buf_v, [rsp, idv], ones, mask=m)
            return carry

        jax.lax.fori_loop(0, _RPW // _L, group, 0)
        pltpu.sync_copy(buf_v, out_hbm.at[pl.ds(base, _RPW)])

    return k(ids_flat, nums_flat, zeros2d)


def _sc_extract(sp2d, cnt_flat, zeros2d, d, kk):
    pad = kk + 2 * _L

    @functools.partial(
        pl.kernel,
        mesh=_sc_mesh(),
        compiler_params=pltpu.CompilerParams(
            needs_layout_passes=False, use_tc_tiling_on_sc=True),
        out_type=jax.ShapeDtypeStruct((_ROWS, kk), jnp.int32),
        scratch_types=[
            pltpu.VMEM((_RPW,), jnp.int32),
            pltpu.VMEM((_RPW, kk), jnp.int32),
            pltpu.VMEM((1, d), jnp.float32),
            pltpu.VMEM((pad,), jnp.int32),
        ],
    )
    def k(sp_hbm, cnt_hbm, z_hbm, out_hbm, cnt_v, buf_v, row_v, ext_v):
        wid = _worker_id()
        base = wid * _RPW
        pltpu.sync_copy(cnt_hbm.at[pl.ds(base, _RPW)], cnt_v)
        pltpu.sync_copy(z_hbm, buf_v)
        lane = jax.lax.iota(jnp.int32, _L)
        zv = jnp.zeros((_L,), jnp.int32)

        def group(g, carry):
            cntv = cnt_v[pl.ds(g * _L, _L)]
            for j in range(_L):
                r = g * _L + j
                cnt = cntv[j]

                @pl.when(cnt > 0)
                def _(r=r):
                    pltpu.sync_copy(sp_hbm.at[pl.ds(base + r, 1)], row_v)
                    for z in range(pad // _L):
                        ext_v[pl.ds(z * _L, _L)] = zv

                    def chunk(c, off):
                        v = row_v[0, pl.ds(c * _L, _L)]
                        m = v > 0.0
                        s = jnp.sum(m.astype(jnp.int32))

                        @pl.when((s > 0) & (off < kk))
                        def _():
                            plsc.store_compressed(
                                ext_v.at[pl.ds(off, _L)], lane + (c * _L),
                                mask=m)

                        return off + s

                    jax.lax.fori_loop(0, d // _L, chunk, 0)
                    for z in range(kk // _L):
                        buf_v[r, pl.ds(z * _L, _L)] = ext_v[pl.ds(z * _L, _L)]

            return carry

        jax.lax.fori_loop(0, _RPW // _L, group, 0)
        pltpu.sync_copy(buf_v, out_hbm.at[pl.ds(base, _RPW)])

    return k(sp2d, cnt_flat, zeros2d)


_CHUNK = 64
_RC = _CHUNK * _B


def _fused_body(dense_ref, w0_ref, w1_ref, w2_ref,
                dec0_ref, thr0_ref, init0_ref,
                dec1_ref, thr1_ref, init1_ref,
                dec2_ref, thr2_ref, init2_ref,
                st0_ref, sp0_ref, n0_ref,
                st1_ref, sp1_ref, n1_ref,
                st2_ref, sp2_ref, n2_ref,
                syn_scr, syn2_scr, v0_scr, v1_scr, v2_scr):
    @pl.when(pl.program_id(0) == 0)
    def _():
        v0_scr[...] = init0_ref[...]
        v1_scr[...] = init1_ref[...]
        v2_scr[...] = init2_ref[...]

    def lif(syn_scr_ref, dec_ref, thr_ref, v_scr, st_ref, sp_ref, n_ref, kk):
        dec = dec_ref[...]
        om = 1.0 - dec
        thr = thr_ref[...]
        zero = jnp.zeros((), jnp.float32)
        one = jnp.ones((), jnp.float32)

        def step(i, carry):
            v, tot = carry
            rs = pl.ds(i * _B, _B)
            v = dec * v + om * syn_scr_ref[rs]
            fired = v > thr
            sp = jnp.where(fired, one, zero)
            v = jnp.where(fired, zero, v)
            st_ref[rs] = v
            sp_ref[rs] = sp
            cnt = jnp.sum(sp, axis=1, keepdims=True)
            n_ref[rs] = jnp.minimum(cnt, float(kk)).astype(jnp.int32)
            return v, tot + jnp.sum(cnt)

        v, tot = jax.lax.fori_loop(
            0, _CHUNK, step, (v_scr[...], jnp.zeros((), jnp.float32)),
            unroll=4)
        v_scr[...] = v
        return tot

    def mm(x, w_ref):
        return jax.lax.dot_general(
            x, w_ref[...], (((1,), (1,)), ((), ())),
            preferred_element_type=jnp.float32)

    syn_scr[...] = mm(dense_ref[...], w0_ref)
    tot0 = lif(syn_scr, dec0_ref, thr0_ref, v0_scr, st0_ref, sp0_ref,
               n0_ref, 128)

    @pl.when(tot0 > 0.0)
    def _():
        syn_scr[...] = mm(sp0_ref[...], w1_ref)

    @pl.when(tot0 == 0.0)
    def _():
        syn_scr[...] = jnp.zeros((_RC, 1024), jnp.float32)

    tot1 = lif(syn_scr, dec1_ref, thr1_ref, v1_scr, st1_ref, sp1_ref,
               n1_ref, 128)

    @pl.when(tot1 > 0.0)
    def _():
        syn2_scr[...] = mm(sp1_ref[...], w2_ref)

    @pl.when(tot1 == 0.0)
    def _():
        syn2_scr[...] = jnp.zeros((_RC, 512), jnp.float32)

    lif(syn2_scr, dec2_ref, thr2_ref, v2_scr, st2_ref, sp2_ref, n2_ref, 64)


def _fused_net(dense, w0, w1, w2, dec0, thr0, init0, dec1, thr1, init1,
               dec2, thr2, init2):
    const = lambda i: (0, 0)
    rowblk = lambda i: (i, 0)
    return pl.pallas_call(
        _fused_body,
        grid=(_ROWS // _RC,),
        in_specs=[
            pl.BlockSpec((_RC, 512), rowblk),
            pl.BlockSpec((1024, 512), const),
            pl.BlockSpec((1024, 1024), const),
            pl.BlockSpec((512, 1024), const),
            pl.BlockSpec((1, 1024), const),
            pl.BlockSpec((1, 1024), const),
            pl.BlockSpec((_B, 1024), const),
            pl.BlockSpec((1, 1024), const),
            pl.BlockSpec((1, 1024), const),
            pl.BlockSpec((_B, 1024), const),
            pl.BlockSpec((1, 512), const),
            pl.BlockSpec((1, 512), const),
            pl.BlockSpec((_B, 512), const),
        ],
        out_specs=[
            pl.BlockSpec((_RC, 1024), rowblk),
            pl.BlockSpec((_RC, 1024), rowblk),
            pl.BlockSpec((_RC, 1), rowblk),
            pl.BlockSpec((_RC, 1024), rowblk),
            pl.BlockSpec((_RC, 1024), rowblk),
            pl.BlockSpec((_RC, 1), rowblk),
            pl.BlockSpec((_RC, 512), rowblk),
            pl.BlockSpec((_RC, 512), rowblk),
            pl.BlockSpec((_RC, 1), rowblk),
        ],
        out_shape=[
            jax.ShapeDtypeStruct((_ROWS, 1024), jnp.float32),
            jax.ShapeDtypeStruct((_ROWS, 1024), jnp.float32),
            jax.ShapeDtypeStruct((_ROWS, 1), jnp.int32),
            jax.ShapeDtypeStruct((_ROWS, 1024), jnp.float32),
            jax.ShapeDtypeStruct((_ROWS, 1024), jnp.float32),
            jax.ShapeDtypeStruct((_ROWS, 1), jnp.int32),
            jax.ShapeDtypeStruct((_ROWS, 512), jnp.float32),
            jax.ShapeDtypeStruct((_ROWS, 512), jnp.float32),
            jax.ShapeDtypeStruct((_ROWS, 1), jnp.int32),
        ],
        scratch_shapes=[
            pltpu.VMEM((_RC, 1024), jnp.float32),
            pltpu.VMEM((_RC, 512), jnp.float32),
            pltpu.VMEM((_B, 1024), jnp.float32),
            pltpu.VMEM((_B, 1024), jnp.float32),
            pltpu.VMEM((_B, 512), jnp.float32),
        ],
    )(dense, w0, w1, w2, dec0.reshape(1, 1024), thr0.reshape(1, 1024), init0,
      dec1.reshape(1, 1024), thr1.reshape(1, 1024), init1,
      dec2.reshape(1, 512), thr2.reshape(1, 512), init2)


def kernel(inp_spike_ids, num_inp_spikes, init_state_0, init_state_1,
           init_state_2, w0, w1, w2, decay_0, decay_1, decay_2,
           thr_0, thr_1, thr_2):
    ids_flat = inp_spike_ids.reshape(_ROWS * _S_IN)
    nums_flat = num_inp_spikes.reshape(_ROWS)
    zf = jnp.zeros((_RPW, 512), jnp.float32)

    dense = _sc_scatter(ids_flat, nums_flat, zf)

    st0, sp0, n0, st1, sp1, n1, st2, sp2, n2 = _fused_net(
        dense, w0, w1, w2, decay_0, thr_0, init_state_0,
        decay_1, thr_1, init_state_1, decay_2, thr_2, init_state_2)

    zi128 = jnp.zeros((_RPW, 128), jnp.int32)
    zi64 = jnp.zeros((_RPW, 64), jnp.int32)
    ids0 = _sc_extract(sp0, n0.reshape(_ROWS), zi128, 1024, 128)
    ids1 = _sc_extract(sp1, n1.reshape(_ROWS), zi128, 1024, 128)
    ids2 = _sc_extract(sp2, n2.reshape(_ROWS), zi64, 512, 64)

    return (ids0.reshape(_SEQ, _B, 128), ids1.reshape(_SEQ, _B, 128),
            ids2.reshape(_SEQ, _B, 64),
            n0.reshape(_SEQ, _B, 1), n1.reshape(_SEQ, _B, 1),
            n2.reshape(_SEQ, _B, 1),
            st0.reshape(_SEQ, _B, 1024), st1.reshape(_SEQ, _B, 1024),
            st2.reshape(_SEQ, _B, 512))


# speedup vs baseline: 1.0812x; 1.0812x over previous
import functools

import jax
import jax.numpy as jnp
from jax.experimental import pallas as pl
from jax.experimental.pallas import tpu as pltpu
from jax.experimental.pallas import tpu_sc as plsc

_SEQ = 512
_B = 8
_ROWS = _SEQ * _B
_NC = 2
_NS = 16
_NW = _NC * _NS
_RPW = _ROWS // _NW
_L = 16
_S_IN = 64


def _sc_mesh():
    return plsc.VectorSubcoreMesh(core_axis_name="c", subcore_axis_name="s")


def _worker_id():
    return jax.lax.axis_index("s") * _NC + jax.lax.axis_index("c")


def _sc_scatter(ids_flat, nums_flat, zeros2d):
    din = 512

    @functools.partial(
        pl.kernel,
        mesh=_sc_mesh(),
        compiler_params=pltpu.CompilerParams(
            needs_layout_passes=False, use_tc_tiling_on_sc=True),
        out_type=jax.ShapeDtypeStruct((_ROWS, din), jnp.float32),
        scratch_types=[
            pltpu.VMEM((_RPW * _S_IN,), jnp.int32),
            pltpu.VMEM((_RPW,), jnp.int32),
            pltpu.VMEM((_RPW, din), jnp.float32),
        ],
    )
    def k(ids_hbm, nums_hbm, z_hbm, out_hbm, ids_v, nums_v, buf_v):
        wid = _worker_id()
        base = wid * _RPW
        pltpu.sync_copy(ids_hbm.at[pl.ds(base * _S_IN, _RPW * _S_IN)], ids_v)
        pltpu.sync_copy(nums_hbm.at[pl.ds(base, _RPW)], nums_v)
        pltpu.sync_copy(z_hbm, buf_v)
        ones = jnp.full((_L,), 1.0, jnp.float32)
        lane = jax.lax.iota(jnp.int32, _L)

        def group(g, carry):
            numv = nums_v[pl.ds(g * _L, _L)]
            for j in range(_L):
                r = g * _L + j
                num = numv[j]
                rsp = jnp.full((_L,), 0, jnp.int32) + r
                for c in range(_S_IN // _L):
                    idv = ids_v[pl.ds(r * _S_IN + c * _L, _L)]
                    m = (lane + (c * _L)) < num
                    plsc.store_scatter(buf_v, [rsp, idv], ones, mask=m)
            return carry

        jax.lax.fori_loop(0, _RPW // _L, group, 0)
        pltpu.sync_copy(buf_v, out_hbm.at[pl.ds(base, _RPW)])

    return k(ids_flat, nums_flat, zeros2d)


def _sc_extract(sp2d, cnt_flat, zeros2d, d, kk):
    pad = kk + 2 * _L

    @functools.partial(
        pl.kernel,
        mesh=_sc_mesh(),
        compiler_params=pltpu.CompilerParams(
            needs_layout_passes=False, use_tc_tiling_on_sc=True),
        out_type=jax.ShapeDtypeStruct((_ROWS, kk), jnp.int32),
        scratch_types=[
            pltpu.VMEM((_RPW,), jnp.int32),
            pltpu.VMEM((_RPW, kk), jnp.int32),
            pltpu.VMEM((1, d), jnp.float32),
            pltpu.VMEM((pad,), jnp.int32),
        ],
    )
    def k(sp_hbm, cnt_hbm, z_hbm, out_hbm, cnt_v, buf_v, row_v, ext_v):
        wid = _worker_id()
        base = wid * _RPW
        pltpu.sync_copy(cnt_hbm.at[pl.ds(base, _RPW)], cnt_v)
        pltpu.sync_copy(z_hbm, buf_v)
        lane = jax.lax.iota(jnp.int32, _L)
        zv = jnp.zeros((_L,), jnp.int32)

        def group(g, carry):
            cntv = cnt_v[pl.ds(g * _L, _L)]
            for j in range(_L):
                r = g * _L + j
                cnt = cntv[j]

                @pl.when(cnt > 0)
                def _(r=r):
                    pltpu.sync_copy(sp_hbm.at[pl.ds(base + r, 1)], row_v)
                    for z in range(pad // _L):
                        ext_v[pl.ds(z * _L, _L)] = zv

                    def chunk(c, off):
                        v = row_v[0, pl.ds(c * _L, _L)]
                        m = v > 0.0
                        s = jnp.sum(m.astype(jnp.int32))

                        @pl.when((s > 0) & (off < kk))
                        def _():
                            plsc.store_compressed(
                                ext_v.at[pl.ds(off, _L)], lane + (c * _L),
                                mask=m)

                        return off + s

                    jax.lax.fori_loop(0, d // _L, chunk, 0)
                    for z in range(kk // _L):
                        buf_v[r, pl.ds(z * _L, _L)] = ext_v[pl.ds(z * _L, _L)]

            return carry

        jax.lax.fori_loop(0, _RPW // _L, group, 0)
        pltpu.sync_copy(buf_v, out_hbm.at[pl.ds(base, _RPW)])

    return k(sp2d, cnt_flat, zeros2d)


_CHUNK = 64
_RC = _CHUNK * _B


def _lif_chunk(syn_scr_ref, dec_ref, thr_ref, v_scr, st_ref, sp_ref, n_ref,
               kk):
    dec = dec_ref[...]
    om = 1.0 - dec
    thr = thr_ref[...]
    zero = jnp.zeros((), jnp.float32)
    one = jnp.ones((), jnp.float32)

    def step(i, v):
        rs = pl.ds(i * _B, _B)
        v = dec * v + om * syn_scr_ref[rs]
        fired = v > thr
        sp = jnp.where(fired, one, zero)
        v = jnp.where(fired, zero, v)
        st_ref[rs] = v
        sp_ref[rs] = sp
        cnt = jnp.sum(sp, axis=1, keepdims=True)
        n_ref[rs] = jnp.minimum(cnt, float(kk)).astype(jnp.int32)
        return v

    v_scr[...] = jax.lax.fori_loop(0, _CHUNK, step, v_scr[...], unroll=4)


def _mm(x, w_ref):
    return jax.lax.dot_general(
        x, w_ref[...], (((1,), (1,)), ((), ())),
        preferred_element_type=jnp.float32)


def _net01_body(dense_ref, w0_ref, w1_ref,
                dec0_ref, thr0_ref, init0_ref,
                dec1_ref, thr1_ref, init1_ref,
                st0_ref, sp0_ref, n0_ref,
                st1_ref, sp1_ref, n1_ref,
                syn_scr, v0_scr, v1_scr):
    @pl.when(pl.program_id(0) == 0)
    def _():
        v0_scr[...] = init0_ref[...]
        v1_scr[...] = init1_ref[...]

    syn_scr[...] = _mm(dense_ref[...], w0_ref)
    _lif_chunk(syn_scr, dec0_ref, thr0_ref, v0_scr, st0_ref, sp0_ref,
               n0_ref, 128)
    syn_scr[...] = _mm(sp0_ref[...], w1_ref)
    _lif_chunk(syn_scr, dec1_ref, thr1_ref, v1_scr, st1_ref, sp1_ref,
               n1_ref, 128)


def _net2_body(sp1_ref, w2_ref, dec2_ref, thr2_ref, init2_ref,
               st2_ref, sp2_ref, n2_ref, syn_scr, v2_scr):
    @pl.when(pl.program_id(0) == 0)
    def _():
        v2_scr[...] = init2_ref[...]

    syn_scr[...] = _mm(sp1_ref[...], w2_ref)
    _lif_chunk(syn_scr, dec2_ref, thr2_ref, v2_scr, st2_ref, sp2_ref,
               n2_ref, 64)


def _net01(dense, w0, w1, dec0, thr0, init0, dec1, thr1, init1):
    const = lambda i: (0, 0)
    rowblk = lambda i: (i, 0)
    return pl.pallas_call(
        _net01_body,
        grid=(_ROWS // _RC,),
        in_specs=[
            pl.BlockSpec((_RC, 512), rowblk),
            pl.BlockSpec((1024, 512), const),
            pl.BlockSpec((1024, 1024), const),
            pl.BlockSpec((1, 1024), const),
            pl.BlockSpec((1, 1024), const),
            pl.BlockSpec((_B, 1024), const),
            pl.BlockSpec((1, 1024), const),
            pl.BlockSpec((1, 1024), const),
            pl.BlockSpec((_B, 1024), const),
        ],
        out_specs=[
            pl.BlockSpec((_RC, 1024), rowblk),
            pl.BlockSpec((_RC, 1024), rowblk),
            pl.BlockSpec((_RC, 1), rowblk),
            pl.BlockSpec((_RC, 1024), rowblk),
            pl.BlockSpec((_RC, 1024), rowblk),
            pl.BlockSpec((_RC, 1), rowblk),
        ],
        out_shape=[
            jax.ShapeDtypeStruct((_ROWS, 1024), jnp.float32),
            jax.ShapeDtypeStruct((_ROWS, 1024), jnp.float32),
            jax.ShapeDtypeStruct((_ROWS, 1), jnp.int32),
            jax.ShapeDtypeStruct((_ROWS, 1024), jnp.float32),
            jax.ShapeDtypeStruct((_ROWS, 1024), jnp.float32),
            jax.ShapeDtypeStruct((_ROWS, 1), jnp.int32),
        ],
        scratch_shapes=[
            pltpu.VMEM((_RC, 1024), jnp.float32),
            pltpu.VMEM((_B, 1024), jnp.float32),
            pltpu.VMEM((_B, 1024), jnp.float32),
        ],
    )(dense, w0, w1, dec0.reshape(1, 1024), thr0.reshape(1, 1024), init0,
      dec1.reshape(1, 1024), thr1.reshape(1, 1024), init1)


def _net2(sp1, w2, dec2, thr2, init2):
    const = lambda i: (0, 0)
    rowblk = lambda i: (i, 0)
    return pl.pallas_call(
        _net2_body,
        grid=(_ROWS // _RC,),
        in_specs=[
            pl.BlockSpec((_RC, 1024), rowblk),
            pl.BlockSpec((512, 1024), const),
            pl.BlockSpec((1, 512), const),
            pl.BlockSpec((1, 512), const),
            pl.BlockSpec((_B, 512), const),
        ],
        out_specs=[
            pl.BlockSpec((_RC, 512), rowblk),
            pl.BlockSpec((_RC, 512), rowblk),
            pl.BlockSpec((_RC, 1), rowblk),
        ],
        out_shape=[
            jax.ShapeDtypeStruct((_ROWS, 512), jnp.float32),
            jax.ShapeDtypeStruct((_ROWS, 512), jnp.float32),
            jax.ShapeDtypeStruct((_ROWS, 1), jnp.int32),
        ],
        scratch_shapes=[
            pltpu.VMEM((_RC, 512), jnp.float32),
            pltpu.VMEM((_B, 512), jnp.float32),
        ],
    )(sp1, w2, dec2.reshape(1, 512), thr2.reshape(1, 512), init2)


def kernel(inp_spike_ids, num_inp_spikes, init_state_0, init_state_1,
           init_state_2, w0, w1, w2, decay_0, decay_1, decay_2,
           thr_0, thr_1, thr_2):
    ids_flat = inp_spike_ids.reshape(_ROWS * _S_IN)
    nums_flat = num_inp_spikes.reshape(_ROWS)
    zf = jnp.zeros((_RPW, 512), jnp.float32)

    dense = _sc_scatter(ids_flat, nums_flat, zf)

    st0, sp0, n0, st1, sp1, n1 = _net01(
        dense, w0, w1, decay_0, thr_0, init_state_0,
        decay_1, thr_1, init_state_1)
    st2, sp2, n2 = _net2(sp1, w2, decay_2, thr_2, init_state_2)

    zi128 = jnp.zeros((_RPW, 128), jnp.int32)
    zi64 = jnp.zeros((_RPW, 64), jnp.int32)
    ids0 = _sc_extract(sp0, n0.reshape(_ROWS), zi128, 1024, 128)
    ids1 = _sc_extract(sp1, n1.reshape(_ROWS), zi128, 1024, 128)
    ids2 = _sc_extract(sp2, n2.reshape(_ROWS), zi64, 512, 64)

    return (ids0.reshape(_SEQ, _B, 128), ids1.reshape(_SEQ, _B, 128),
            ids2.reshape(_SEQ, _B, 64),
            n0.reshape(_SEQ, _B, 1), n1.reshape(_SEQ, _B, 1),
            n2.reshape(_SEQ, _B, 1),
            st0.reshape(_SEQ, _B, 1024), st1.reshape(_SEQ, _B, 1024),
            st2.reshape(_SEQ, _B, 512))


# speedup vs baseline: 1.0831x; 1.0017x over previous
import functools

import jax
import jax.numpy as jnp
from jax.experimental import pallas as pl
from jax.experimental.pallas import tpu as pltpu
from jax.experimental.pallas import tpu_sc as plsc

_SEQ = 512
_B = 8
_ROWS = _SEQ * _B
_NC = 2
_NS = 16
_NW = _NC * _NS
_RPW = _ROWS // _NW
_L = 16
_S_IN = 64


def _sc_mesh():
    return plsc.VectorSubcoreMesh(core_axis_name="c", subcore_axis_name="s")


def _worker_id():
    return jax.lax.axis_index("s") * _NC + jax.lax.axis_index("c")


def _sc_scatter(ids_flat, nums_flat, zeros2d):
    din = 512

    @functools.partial(
        pl.kernel,
        mesh=_sc_mesh(),
        compiler_params=pltpu.CompilerParams(
            needs_layout_passes=False, use_tc_tiling_on_sc=True),
        out_type=jax.ShapeDtypeStruct((_ROWS, din), jnp.float32),
        scratch_types=[
            pltpu.VMEM((_RPW * _S_IN,), jnp.int32),
            pltpu.VMEM((_RPW,), jnp.int32),
            pltpu.VMEM((_RPW, din), jnp.float32),
        ],
    )
    def k(ids_hbm, nums_hbm, z_hbm, out_hbm, ids_v, nums_v, buf_v):
        wid = _worker_id()
        base = wid * _RPW
        pltpu.sync_copy(ids_hbm.at[pl.ds(base * _S_IN, _RPW * _S_IN)], ids_v)
        pltpu.sync_copy(nums_hbm.at[pl.ds(base, _RPW)], nums_v)
        pltpu.sync_copy(z_hbm, buf_v)
        ones = jnp.full((_L,), 1.0, jnp.float32)
        lane = jax.lax.iota(jnp.int32, _L)

        def group(g, carry):
            numv = nums_v[pl.ds(g * _L, _L)]
            for j in range(_L):
                r = g * _L + j
                num = numv[j]
                rsp = jnp.full((_L,), 0, jnp.int32) + r
                for c in range(_S_IN // _L):
                    idv = ids_v[pl.ds(r * _S_IN + c * _L, _L)]
                    m = (lane + (c * _L)) < num
                    plsc.store_scatter(buf_v, [rsp, idv], ones, mask=m)
            return carry

        jax.lax.fori_loop(0, _RPW // _L, group, 0)
        pltpu.sync_copy(buf_v, out_hbm.at[pl.ds(base, _RPW)])

    return k(ids_flat, nums_flat, zeros2d)


def _sc_extract(sp2d, cnt_flat, zeros2d, d, kk):
    pad = kk + 2 * _L

    @functools.partial(
        pl.kernel,
        mesh=_sc_mesh(),
        compiler_params=pltpu.CompilerParams(
            needs_layout_passes=False, use_tc_tiling_on_sc=True),
        out_type=jax.ShapeDtypeStruct((_ROWS, kk), jnp.int32),
        scratch_types=[
            pltpu.VMEM((_RPW,), jnp.int32),
            pltpu.VMEM((_RPW, kk), jnp.int32),
            pltpu.VMEM((1, d), jnp.float32),
            pltpu.VMEM((pad,), jnp.int32),
        ],
    )
    def k(sp_hbm, cnt_hbm, z_hbm, out_hbm, cnt_v, buf_v, row_v, ext_v):
        wid = _worker_id()
        base = wid * _RPW
        pltpu.sync_copy(cnt_hbm.at[pl.ds(base, _RPW)], cnt_v)
        pltpu.sync_copy(z_hbm, buf_v)
        lane = jax.lax.iota(jnp.int32, _L)
        zv = jnp.zeros((_L,), jnp.int32)

        def group(g, carry):
            cntv = cnt_v[pl.ds(g * _L, _L)]
            for j in range(_L):
                r = g * _L + j
                cnt = cntv[j]

                @pl.when(cnt > 0)
                def _(r=r):
                    pltpu.sync_copy(sp_hbm.at[pl.ds(base + r, 1)], row_v)
                    for z in range(pad // _L):
                        ext_v[pl.ds(z * _L, _L)] = zv

                    def chunk(c, off):
                        v = row_v[0, pl.ds(c * _L, _L)]
                        m = v > 0.0
                        s = jnp.sum(m.astype(jnp.int32))

                        @pl.when((s > 0) & (off < kk))
                        def _():
                            plsc.store_compressed(
                                ext_v.at[pl.ds(off, _L)], lane + (c * _L),
                                mask=m)

                        return off + s

                    jax.lax.fori_loop(0, d // _L, chunk, 0)
                    for z in range(kk // _L):
                        buf_v[r, pl.ds(z * _L, _L)] = ext_v[pl.ds(z * _L, _L)]

            return carry

        jax.lax.fori_loop(0, _RPW // _L, group, 0)
        pltpu.sync_copy(buf_v, out_hbm.at[pl.ds(base, _RPW)])

    return k(sp2d, cnt_flat, zeros2d)


_CHUNK = 64
_RC = _CHUNK * _B


def _lif_chunk(syn_scr_ref, dec_ref, thr_ref, v_scr, st_ref, sp_ref, n_ref,
               kk):
    dec = dec_ref[...]
    om = 1.0 - dec
    thr = thr_ref[...]
    zero = jnp.zeros((), jnp.float32)
    one = jnp.ones((), jnp.float32)

    def step(i, v):
        rs = pl.ds(i * _B, _B)
        v = dec * v + om * syn_scr_ref[rs]
        fired = v > thr
        sp = jnp.where(fired, one, zero)
        v = jnp.where(fired, zero, v)
        st_ref[rs] = v
        sp_ref[rs] = sp
        cnt = jnp.sum(sp, axis=1, keepdims=True)
        n_ref[rs] = jnp.minimum(cnt, float(kk)).astype(jnp.int32)
        return v

    v_scr[...] = jax.lax.fori_loop(0, _CHUNK, step, v_scr[...], unroll=4)


def _mm(x, w_ref):
    return jax.lax.dot_general(
        x, w_ref[...], (((1,), (1,)), ((), ())),
        preferred_element_type=jnp.float32)


def _net01_body(dense_ref, w0_ref, w1_ref,
                dec0_ref, thr0_ref, init0_ref,
                dec1_ref, thr1_ref, init1_ref,
                st0_ref, sp0_ref, n0_ref,
                st1_ref, sp1_ref, n1_ref,
                syn_scr, v0_scr, v1_scr):
    @pl.when(pl.program_id(0) == 0)
    def _():
        v0_scr[...] = init0_ref[...]
        v1_scr[...] = init1_ref[...]

    syn_scr[...] = _mm(dense_ref[...], w0_ref)
    _lif_chunk(syn_scr, dec0_ref, thr0_ref, v0_scr, st0_ref, sp0_ref,
               n0_ref, 128)
    syn_scr[...] = _mm(sp0_ref[...], w1_ref)
    _lif_chunk(syn_scr, dec1_ref, thr1_ref, v1_scr, st1_ref, sp1_ref,
               n1_ref, 128)


def _net2_body(sp1_ref, w2_ref, dec2_ref, thr2_ref, init2_ref,
               st2_ref, sp2_ref, n2_ref, syn_scr, v2_scr):
    @pl.when(pl.program_id(0) == 0)
    def _():
        v2_scr[...] = init2_ref[...]

    syn_scr[...] = _mm(sp1_ref[...], w2_ref)
    _lif_chunk(syn_scr, dec2_ref, thr2_ref, v2_scr, st2_ref, sp2_ref,
               n2_ref, 64)


def _net01(dense, w0, w1, dec0, thr0, init0, dec1, thr1, init1):
    const = lambda i: (0, 0)
    rowblk = lambda i: (i, 0)
    return pl.pallas_call(
        _net01_body,
        grid=(_ROWS // _RC,),
        in_specs=[
            pl.BlockSpec((_RC, 512), rowblk),
            pl.BlockSpec((1024, 512), const),
            pl.BlockSpec((1024, 1024), const),
            pl.BlockSpec((1, 1024), const),
            pl.BlockSpec((1, 1024), const),
            pl.BlockSpec((_B, 1024), const),
            pl.BlockSpec((1, 1024), const),
            pl.BlockSpec((1, 1024), const),
            pl.BlockSpec((_B, 1024), const),
        ],
        out_specs=[
            pl.BlockSpec((_RC, 1024), rowblk),
            pl.BlockSpec((_RC, 1024), rowblk),
            pl.BlockSpec((_RC, 1), rowblk),
            pl.BlockSpec((_RC, 1024), rowblk),
            pl.BlockSpec((_RC, 1024), rowblk),
            pl.BlockSpec((_RC, 1), rowblk),
        ],
        out_shape=[
            jax.ShapeDtypeStruct((_ROWS, 1024), jnp.float32),
            jax.ShapeDtypeStruct((_ROWS, 1024), jnp.float32),
            jax.ShapeDtypeStruct((_ROWS, 1), jnp.int32),
            jax.ShapeDtypeStruct((_ROWS, 1024), jnp.float32),
            jax.ShapeDtypeStruct((_ROWS, 1024), jnp.float32),
            jax.ShapeDtypeStruct((_ROWS, 1), jnp.int32),
        ],
        scratch_shapes=[
            pltpu.VMEM((_RC, 1024), jnp.float32),
            pltpu.VMEM((_B, 1024), jnp.float32),
            pltpu.VMEM((_B, 1024), jnp.float32),
        ],
    )(dense, w0, w1, dec0.reshape(1, 1024), thr0.reshape(1, 1024), init0,
      dec1.reshape(1, 1024), thr1.reshape(1, 1024), init1)


def _net2(sp1, w2, dec2, thr2, init2):
    const = lambda i: (0, 0)
    rowblk = lambda i: (i, 0)
    return pl.pallas_call(
        _net2_body,
        grid=(_ROWS // _RC,),
        in_specs=[
            pl.BlockSpec((_RC, 1024), rowblk),
            pl.BlockSpec((512, 1024), const),
            pl.BlockSpec((1, 512), const),
            pl.BlockSpec((1, 512), const),
            pl.BlockSpec((_B, 512), const),
        ],
        out_specs=[
            pl.BlockSpec((_RC, 512), rowblk),
            pl.BlockSpec((_RC, 512), rowblk),
            pl.BlockSpec((_RC, 1), rowblk),
        ],
        out_shape=[
            jax.ShapeDtypeStruct((_ROWS, 512), jnp.float32),
            jax.ShapeDtypeStruct((_ROWS, 512), jnp.float32),
            jax.ShapeDtypeStruct((_ROWS, 1), jnp.int32),
        ],
        scratch_shapes=[
            pltpu.VMEM((_RC, 512), jnp.float32),
            pltpu.VMEM((_B, 512), jnp.float32),
        ],
    )(sp1, w2, dec2.reshape(1, 512), thr2.reshape(1, 512), init2)


def kernel(inp_spike_ids, num_inp_spikes, init_state_0, init_state_1,
           init_state_2, w0, w1, w2, decay_0, decay_1, decay_2,
           thr_0, thr_1, thr_2):
    ids_flat = inp_spike_ids.reshape(_ROWS * _S_IN)
    nums_flat = num_inp_spikes.reshape(_ROWS)
    zf = jnp.zeros((_RPW, 512), jnp.float32)

    dense = _sc_scatter(ids_flat, nums_flat, zf)

    st0, sp0, n0, st1, sp1, n1 = _net01(
        dense, w0, w1, decay_0, thr_0, init_state_0,
        decay_1, thr_1, init_state_1)

    zi128 = jnp.zeros((_RPW, 128), jnp.int32)
    zi64 = jnp.zeros((_RPW, 64), jnp.int32)
    ids0 = _sc_extract(sp0, n0.reshape(_ROWS), zi128, 1024, 128)
    ids1 = _sc_extract(sp1, n1.reshape(_ROWS), zi128, 1024, 128)
    st2, sp2, n2 = _net2(sp1, w2, decay_2, thr_2, init_state_2)
    ids2 = _sc_extract(sp2, n2.reshape(_ROWS), zi64, 512, 64)

    return (ids0.reshape(_SEQ, _B, 128), ids1.reshape(_SEQ, _B, 128),
            ids2.reshape(_SEQ, _B, 64),
            n0.reshape(_SEQ, _B, 1), n1.reshape(_SEQ, _B, 1),
            n2.reshape(_SEQ, _B, 1),
            st0.reshape(_SEQ, _B, 1024), st1.reshape(_SEQ, _B, 1024),
            st2.reshape(_SEQ, _B, 512))


# speedup vs baseline: 1.2264x; 1.1323x over previous
import functools

import jax
import jax.numpy as jnp
from jax.experimental import pallas as pl
from jax.experimental.pallas import tpu as pltpu
from jax.experimental.pallas import tpu_sc as plsc

_SEQ = 512
_B = 8
_ROWS = _SEQ * _B
_NC = 2
_NS = 16
_NW = _NC * _NS
_RPW = _ROWS // _NW
_L = 16
_S_IN = 64


def _sc_mesh():
    return plsc.VectorSubcoreMesh(core_axis_name="c", subcore_axis_name="s")


def _worker_id():
    return jax.lax.axis_index("s") * _NC + jax.lax.axis_index("c")


def _sc_scatter(ids_flat, nums_flat, zeros2d):
    din = 512

    @functools.partial(
        pl.kernel,
        mesh=_sc_mesh(),
        compiler_params=pltpu.CompilerParams(
            needs_layout_passes=False, use_tc_tiling_on_sc=True),
        out_type=jax.ShapeDtypeStruct((_ROWS, din), jnp.float32),
        scratch_types=[
            pltpu.VMEM((_RPW * _S_IN,), jnp.int32),
            pltpu.VMEM((_RPW,), jnp.int32),
            pltpu.VMEM((_RPW, din), jnp.float32),
        ],
    )
    def k(ids_hbm, nums_hbm, z_hbm, out_hbm, ids_v, nums_v, buf_v):
        wid = _worker_id()
        base = wid * _RPW
        pltpu.sync_copy(ids_hbm.at[pl.ds(base * _S_IN, _RPW * _S_IN)], ids_v)
        pltpu.sync_copy(nums_hbm.at[pl.ds(base, _RPW)], nums_v)
        pltpu.sync_copy(z_hbm, buf_v)
        ones = jnp.full((_L,), 1.0, jnp.float32)
        lane = jax.lax.iota(jnp.int32, _L)

        def group(g, carry):
            numv = nums_v[pl.ds(g * _L, _L)]
            for j in range(_L):
                r = g * _L + j
                num = numv[j]
                rsp = jnp.full((_L,), 0, jnp.int32) + r
                for c in range(_S_IN // _L):
                    idv = ids_v[pl.ds(r * _S_IN + c * _L, _L)]
                    m = (lane + (c * _L)) < num
                    plsc.store_scatter(buf_v, [rsp, idv], ones, mask=m)
            return carry

        jax.lax.fori_loop(0, _RPW // _L, group, 0)
        pltpu.sync_copy(buf_v, out_hbm.at[pl.ds(base, _RPW)])

    return k(ids_flat, nums_flat, zeros2d)


def _sc_extract(sp2d, cnt_flat, zeros2d, d, kk):
    pad = kk + 2 * _L

    @functools.partial(
        pl.kernel,
        mesh=_sc_mesh(),
        compiler_params=pltpu.CompilerParams(
            needs_layout_passes=False, use_tc_tiling_on_sc=True),
        out_type=jax.ShapeDtypeStruct((_ROWS, kk), jnp.int32),
        scratch_types=[
            pltpu.VMEM((_RPW,), jnp.int32),
            pltpu.VMEM((_RPW, kk), jnp.int32),
            pltpu.VMEM((1, d), jnp.float32),
            pltpu.VMEM((pad,), jnp.int32),
        ],
    )
    def k(sp_hbm, cnt_hbm, z_hbm, out_hbm, cnt_v, buf_v, row_v, ext_v):
        wid = _worker_id()
        base = wid * _RPW
        pltpu.sync_copy(cnt_hbm.at[pl.ds(base, _RPW)], cnt_v)
        pltpu.sync_copy(z_hbm, buf_v)
        lane = jax.lax.iota(jnp.int32, _L)
        zv = jnp.zeros((_L,), jnp.int32)

        def group(g, carry):
            cntv = cnt_v[pl.ds(g * _L, _L)]
            for j in range(_L):
                r = g * _L + j
                cnt = cntv[j]

                @pl.when(cnt > 0)
                def _(r=r):
                    pltpu.sync_copy(sp_hbm.at[pl.ds(base + r, 1)], row_v)
                    for z in range(pad // _L):
                        ext_v[pl.ds(z * _L, _L)] = zv

                    def chunk(c, off):
                        v = row_v[0, pl.ds(c * _L, _L)]
                        m = v > 0.0
                        s = jnp.sum(m.astype(jnp.int32))

                        @pl.when((s > 0) & (off < kk))
                        def _():
                            plsc.store_compressed(
                                ext_v.at[pl.ds(off, _L)], lane + (c * _L),
                                mask=m)

                        return off + s

                    jax.lax.fori_loop(0, d // _L, chunk, 0)
                    for z in range(kk // _L):
                        buf_v[r, pl.ds(z * _L, _L)] = ext_v[pl.ds(z * _L, _L)]

            return carry

        jax.lax.fori_loop(0, _RPW // _L, group, 0)
        pltpu.sync_copy(buf_v, out_hbm.at[pl.ds(base, _RPW)])

    return k(sp2d, cnt_flat, zeros2d)


_CHUNK = 64
_RC = _CHUNK * _B


def _lif_chunk(syn_scr_ref, dec_ref, thr_ref, v_scr, st_ref, sp_ref, n_ref,
               kk):
    dec = dec_ref[...]
    om = 1.0 - dec
    thr = thr_ref[...]
    zero = jnp.zeros((), jnp.float32)
    one = jnp.ones((), jnp.float32)

    def step(i, v):
        rs = pl.ds(i * _B, _B)
        v = dec * v + om * syn_scr_ref[rs]
        fired = v > thr
        sp = jnp.where(fired, one, zero)
        v = jnp.where(fired, zero, v)
        st_ref[rs] = v
        sp_ref[rs] = sp
        cnt = jnp.sum(sp, axis=1, keepdims=True)
        n_ref[rs] = jnp.minimum(cnt, float(kk)).astype(jnp.int32)
        return v

    v_scr[...] = jax.lax.fori_loop(0, _CHUNK, step, v_scr[...], unroll=8)


def _mm(x, w_ref):
    return jax.lax.dot_general(
        x, w_ref[...], (((1,), (1,)), ((), ())),
        preferred_element_type=jnp.float32)


def _fused_body(dense_ref, w0_ref, w1_ref, w2_ref,
                dec0_ref, thr0_ref, init0_ref,
                dec1_ref, thr1_ref, init1_ref,
                dec2_ref, thr2_ref, init2_ref,
                st0_ref, sp0_ref, n0_ref,
                st1_ref, sp1_ref, n1_ref,
                st2_ref, sp2_ref, n2_ref,
                syn_scr, syn2_scr, v0_scr, v1_scr, v2_scr):
    @pl.when(pl.program_id(0) == 0)
    def _():
        v0_scr[...] = init0_ref[...]
        v1_scr[...] = init1_ref[...]
        v2_scr[...] = init2_ref[...]

    syn_scr[...] = _mm(dense_ref[...], w0_ref)
    _lif_chunk(syn_scr, dec0_ref, thr0_ref, v0_scr, st0_ref, sp0_ref,
               n0_ref, 128)
    syn_scr[...] = _mm(sp0_ref[...], w1_ref)
    _lif_chunk(syn_scr, dec1_ref, thr1_ref, v1_scr, st1_ref, sp1_ref,
               n1_ref, 128)
    syn2_scr[...] = _mm(sp1_ref[...], w2_ref)
    _lif_chunk(syn2_scr, dec2_ref, thr2_ref, v2_scr, st2_ref, sp2_ref,
               n2_ref, 64)


def _fused_net(dense, w0, w1, w2, dec0, thr0, init0, dec1, thr1, init1,
               dec2, thr2, init2):
    const = lambda i: (0, 0)
    rowblk = lambda i: (i, 0)
    return pl.pallas_call(
        _fused_body,
        grid=(_ROWS // _RC,),
        in_specs=[
            pl.BlockSpec((_RC, 512), rowblk),
            pl.BlockSpec((1024, 512), const),
            pl.BlockSpec((1024, 1024), const),
            pl.BlockSpec((512, 1024), const),
            pl.BlockSpec((1, 1024), const),
            pl.BlockSpec((1, 1024), const),
            pl.BlockSpec((_B, 1024), const),
            pl.BlockSpec((1, 1024), const),
            pl.BlockSpec((1, 1024), const),
            pl.BlockSpec((_B, 1024), const),
            pl.BlockSpec((1, 512), const),
            pl.BlockSpec((1, 512), const),
            pl.BlockSpec((_B, 512), const),
        ],
        out_specs=[
            pl.BlockSpec((_RC, 1024), rowblk),
            pl.BlockSpec((_RC, 1024), rowblk),
            pl.BlockSpec((_RC, 1), rowblk),
            pl.BlockSpec((_RC, 1024), rowblk),
            pl.BlockSpec((_RC, 1024), rowblk),
            pl.BlockSpec((_RC, 1), rowblk),
            pl.BlockSpec((_RC, 512), rowblk),
            pl.BlockSpec((_RC, 512), rowblk),
            pl.BlockSpec((_RC, 1), rowblk),
        ],
        out_shape=[
            jax.ShapeDtypeStruct((_ROWS, 1024), jnp.float32),
            jax.ShapeDtypeStruct((_ROWS, 1024), jnp.float32),
            jax.ShapeDtypeStruct((_ROWS, 1), jnp.int32),
            jax.ShapeDtypeStruct((_ROWS, 1024), jnp.float32),
            jax.ShapeDtypeStruct((_ROWS, 1024), jnp.float32),
            jax.ShapeDtypeStruct((_ROWS, 1), jnp.int32),
            jax.ShapeDtypeStruct((_ROWS, 512), jnp.float32),
            jax.ShapeDtypeStruct((_ROWS, 512), jnp.float32),
            jax.ShapeDtypeStruct((_ROWS, 1), jnp.int32),
        ],
        scratch_shapes=[
            pltpu.VMEM((_RC, 1024), jnp.float32),
            pltpu.VMEM((_RC, 512), jnp.float32),
            pltpu.VMEM((_B, 1024), jnp.float32),
            pltpu.VMEM((_B, 1024), jnp.float32),
            pltpu.VMEM((_B, 512), jnp.float32),
        ],
    )(dense, w0, w1, w2, dec0.reshape(1, 1024), thr0.reshape(1, 1024), init0,
      dec1.reshape(1, 1024), thr1.reshape(1, 1024), init1,
      dec2.reshape(1, 512), thr2.reshape(1, 512), init2)


def kernel(inp_spike_ids, num_inp_spikes, init_state_0, init_state_1,
           init_state_2, w0, w1, w2, decay_0, decay_1, decay_2,
           thr_0, thr_1, thr_2):
    ids_flat = inp_spike_ids.reshape(_ROWS * _S_IN)
    nums_flat = num_inp_spikes.reshape(_ROWS)
    zf = jnp.zeros((_RPW, 512), jnp.float32)

    dense = _sc_scatter(ids_flat, nums_flat, zf)

    st0, sp0, n0, st1, sp1, n1, st2, sp2, n2 = _fused_net(
        dense, w0, w1, w2, decay_0, thr_0, init_state_0,
        decay_1, thr_1, init_state_1, decay_2, thr_2, init_state_2)

    zi128 = jnp.zeros((_RPW, 128), jnp.int32)
    zi64 = jnp.zeros((_RPW, 64), jnp.int32)
    ids0 = _sc_extract(sp0, n0.reshape(_ROWS), zi128, 1024, 128)
    ids1 = _sc_extract(sp1, n1.reshape(_ROWS), zi128, 1024, 128)
    ids2 = _sc_extract(sp2, n2.reshape(_ROWS), zi64, 512, 64)

    return (ids0.reshape(_SEQ, _B, 128), ids1.reshape(_SEQ, _B, 128),
            ids2.reshape(_SEQ, _B, 64),
            n0.reshape(_SEQ, _B, 1), n1.reshape(_SEQ, _B, 1),
            n2.reshape(_SEQ, _B, 1),
            st0.reshape(_SEQ, _B, 1024), st1.reshape(_SEQ, _B, 1024),
            st2.reshape(_SEQ, _B, 512))


# speedup vs baseline: 1.3051x; 1.0642x over previous
import functools

import jax
import jax.numpy as jnp
from jax.experimental import pallas as pl
from jax.experimental.pallas import tpu as pltpu
from jax.experimental.pallas import tpu_sc as plsc

_SEQ = 512
_B = 8
_ROWS = _SEQ * _B
_NC = 2
_NS = 16
_NW = _NC * _NS
_RPW = _ROWS // _NW
_L = 16
_S_IN = 64


def _sc_mesh():
    return plsc.VectorSubcoreMesh(core_axis_name="c", subcore_axis_name="s")


def _worker_id():
    return jax.lax.axis_index("s") * _NC + jax.lax.axis_index("c")


def _sc_scatter(ids_flat, nums_flat, zeros2d):
    din = 512

    @functools.partial(
        pl.kernel,
        mesh=_sc_mesh(),
        compiler_params=pltpu.CompilerParams(
            needs_layout_passes=False, use_tc_tiling_on_sc=True),
        out_type=jax.ShapeDtypeStruct((_ROWS, din), jnp.float32),
        scratch_types=[
            pltpu.VMEM((_RPW * _S_IN,), jnp.int32),
            pltpu.VMEM((_RPW,), jnp.int32),
            pltpu.VMEM((_RPW, din), jnp.float32),
        ],
    )
    def k(ids_hbm, nums_hbm, z_hbm, out_hbm, ids_v, nums_v, buf_v):
        wid = _worker_id()
        base = wid * _RPW
        pltpu.sync_copy(ids_hbm.at[pl.ds(base * _S_IN, _RPW * _S_IN)], ids_v)
        pltpu.sync_copy(nums_hbm.at[pl.ds(base, _RPW)], nums_v)
        pltpu.sync_copy(z_hbm, buf_v)
        ones = jnp.full((_L,), 1.0, jnp.float32)
        lane = jax.lax.iota(jnp.int32, _L)

        def group(g, carry):
            numv = nums_v[pl.ds(g * _L, _L)]
            for j in range(_L):
                r = g * _L + j
                num = numv[j]
                rsp = jnp.full((_L,), 0, jnp.int32) + r
                for c in range(_S_IN // _L):
                    idv = ids_v[pl.ds(r * _S_IN + c * _L, _L)]
                    m = (lane + (c * _L)) < num
                    plsc.store_scatter(buf_v, [rsp, idv], ones, mask=m)
            return carry

        jax.lax.fori_loop(0, _RPW // _L, group, 0)
        pltpu.sync_copy(buf_v, out_hbm.at[pl.ds(base, _RPW)])

    return k(ids_flat, nums_flat, zeros2d)


def _sc_extract(sp2d, cnt_flat, zeros2d, d, kk):
    pad = kk + 2 * _L

    @functools.partial(
        pl.kernel,
        mesh=_sc_mesh(),
        compiler_params=pltpu.CompilerParams(
            needs_layout_passes=False, use_tc_tiling_on_sc=True),
        out_type=jax.ShapeDtypeStruct((_ROWS, kk), jnp.int32),
        scratch_types=[
            pltpu.VMEM((_RPW,), jnp.int32),
            pltpu.VMEM((_RPW, kk), jnp.int32),
            pltpu.VMEM((1, d), jnp.float32),
            pltpu.VMEM((pad,), jnp.int32),
        ],
    )
    def k(sp_hbm, cnt_hbm, z_hbm, out_hbm, cnt_v, buf_v, row_v, ext_v):
        wid = _worker_id()
        base = wid * _RPW
        pltpu.sync_copy(cnt_hbm.at[pl.ds(base, _RPW)], cnt_v)
        pltpu.sync_copy(z_hbm, buf_v)
        lane = jax.lax.iota(jnp.int32, _L)
        zv = jnp.zeros((_L,), jnp.int32)

        def group(g, carry):
            cntv = cnt_v[pl.ds(g * _L, _L)]
            for j in range(_L):
                r = g * _L + j
                cnt = cntv[j]

                @pl.when(cnt > 0)
                def _(r=r):
                    pltpu.sync_copy(sp_hbm.at[pl.ds(base + r, 1)], row_v)
                    for z in range(pad // _L):
                        ext_v[pl.ds(z * _L, _L)] = zv

                    def chunk(c, off):
                        v = row_v[0, pl.ds(c * _L, _L)]
                        m = v > 0.0
                        s = jnp.sum(m.astype(jnp.int32))

                        @pl.when((s > 0) & (off < kk))
                        def _():
                            plsc.store_compressed(
                                ext_v.at[pl.ds(off, _L)], lane + (c * _L),
                                mask=m)

                        return off + s

                    jax.lax.fori_loop(0, d // _L, chunk, 0)
                    for z in range(kk // _L):
                        buf_v[r, pl.ds(z * _L, _L)] = ext_v[pl.ds(z * _L, _L)]

            return carry

        jax.lax.fori_loop(0, _RPW // _L, group, 0)
        pltpu.sync_copy(buf_v, out_hbm.at[pl.ds(base, _RPW)])

    return k(sp2d, cnt_flat, zeros2d)


_CHUNK = 64
_RC = _CHUNK * _B


def _lif_chunk(syn_scr_ref, dec_ref, thr_ref, v_scr, st_ref, sp_ref, n_ref,
               kk):
    dec = dec_ref[...]
    om = 1.0 - dec
    thr = thr_ref[...]
    zero = jnp.zeros((), jnp.float32)
    one = jnp.ones((), jnp.float32)

    def step(i, v):
        rs = pl.ds(i * _B, _B)
        v = dec * v + om * syn_scr_ref[rs]
        fired = v > thr
        sp = jnp.where(fired, one, zero)
        v = jnp.where(fired, zero, v)
        st_ref[rs] = v
        sp_ref[rs] = sp
        cnt = jnp.sum(sp, axis=1, keepdims=True)
        n_ref[rs] = jnp.minimum(cnt, float(kk)).astype(jnp.int32)
        return v

    v_scr[...] = jax.lax.fori_loop(0, _CHUNK, step, v_scr[...], unroll=16)


def _mm(x, w_ref):
    return jax.lax.dot_general(
        x, w_ref[...], (((1,), (1,)), ((), ())),
        preferred_element_type=jnp.float32)


def _fused_body(dense_ref, w0_ref, w1_ref, w2_ref,
                dec0_ref, thr0_ref, init0_ref,
                dec1_ref, thr1_ref, init1_ref,
                dec2_ref, thr2_ref, init2_ref,
                st0_ref, sp0_ref, n0_ref,
                st1_ref, sp1_ref, n1_ref,
                st2_ref, sp2_ref, n2_ref,
                syn_scr, syn2_scr, v0_scr, v1_scr, v2_scr):
    @pl.when(pl.program_id(0) == 0)
    def _():
        v0_scr[...] = init0_ref[...]
        v1_scr[...] = init1_ref[...]
        v2_scr[...] = init2_ref[...]

    syn_scr[...] = _mm(dense_ref[...], w0_ref)
    _lif_chunk(syn_scr, dec0_ref, thr0_ref, v0_scr, st0_ref, sp0_ref,
               n0_ref, 128)
    syn_scr[...] = _mm(sp0_ref[...], w1_ref)
    _lif_chunk(syn_scr, dec1_ref, thr1_ref, v1_scr, st1_ref, sp1_ref,
               n1_ref, 128)
    syn2_scr[...] = _mm(sp1_ref[...], w2_ref)
    _lif_chunk(syn2_scr, dec2_ref, thr2_ref, v2_scr, st2_ref, sp2_ref,
               n2_ref, 64)


def _fused_net(dense, w0, w1, w2, dec0, thr0, init0, dec1, thr1, init1,
               dec2, thr2, init2):
    const = lambda i: (0, 0)
    rowblk = lambda i: (i, 0)
    return pl.pallas_call(
        _fused_body,
        grid=(_ROWS // _RC,),
        in_specs=[
            pl.BlockSpec((_RC, 512), rowblk),
            pl.BlockSpec((1024, 512), const),
            pl.BlockSpec((1024, 1024), const),
            pl.BlockSpec((512, 1024), const),
            pl.BlockSpec((1, 1024), const),
            pl.BlockSpec((1, 1024), const),
            pl.BlockSpec((_B, 1024), const),
            pl.BlockSpec((1, 1024), const),
            pl.BlockSpec((1, 1024), const),
            pl.BlockSpec((_B, 1024), const),
            pl.BlockSpec((1, 512), const),
            pl.BlockSpec((1, 512), const),
            pl.BlockSpec((_B, 512), const),
        ],
        out_specs=[
            pl.BlockSpec((_RC, 1024), rowblk),
            pl.BlockSpec((_RC, 1024), rowblk),
            pl.BlockSpec((_RC, 1), rowblk),
            pl.BlockSpec((_RC, 1024), rowblk),
            pl.BlockSpec((_RC, 1024), rowblk),
            pl.BlockSpec((_RC, 1), rowblk),
            pl.BlockSpec((_RC, 512), rowblk),
            pl.BlockSpec((_RC, 512), rowblk),
            pl.BlockSpec((_RC, 1), rowblk),
        ],
        out_shape=[
            jax.ShapeDtypeStruct((_ROWS, 1024), jnp.float32),
            jax.ShapeDtypeStruct((_ROWS, 1024), jnp.float32),
            jax.ShapeDtypeStruct((_ROWS, 1), jnp.int32),
            jax.ShapeDtypeStruct((_ROWS, 1024), jnp.float32),
            jax.ShapeDtypeStruct((_ROWS, 1024), jnp.float32),
            jax.ShapeDtypeStruct((_ROWS, 1), jnp.int32),
            jax.ShapeDtypeStruct((_ROWS, 512), jnp.float32),
            jax.ShapeDtypeStruct((_ROWS, 512), jnp.float32),
            jax.ShapeDtypeStruct((_ROWS, 1), jnp.int32),
        ],
        scratch_shapes=[
            pltpu.VMEM((_RC, 1024), jnp.float32),
            pltpu.VMEM((_RC, 512), jnp.float32),
            pltpu.VMEM((_B, 1024), jnp.float32),
            pltpu.VMEM((_B, 1024), jnp.float32),
            pltpu.VMEM((_B, 512), jnp.float32),
        ],
    )(dense, w0, w1, w2, dec0.reshape(1, 1024), thr0.reshape(1, 1024), init0,
      dec1.reshape(1, 1024), thr1.reshape(1, 1024), init1,
      dec2.reshape(1, 512), thr2.reshape(1, 512), init2)


def kernel(inp_spike_ids, num_inp_spikes, init_state_0, init_state_1,
           init_state_2, w0, w1, w2, decay_0, decay_1, decay_2,
           thr_0, thr_1, thr_2):
    ids_flat = inp_spike_ids.reshape(_ROWS * _S_IN)
    nums_flat = num_inp_spikes.reshape(_ROWS)
    zf = jnp.zeros((_RPW, 512), jnp.float32)

    dense = _sc_scatter(ids_flat, nums_flat, zf)

    st0, sp0, n0, st1, sp1, n1, st2, sp2, n2 = _fused_net(
        dense, w0, w1, w2, decay_0, thr_0, init_state_0,
        decay_1, thr_1, init_state_1, decay_2, thr_2, init_state_2)

    zi128 = jnp.zeros((_RPW, 128), jnp.int32)
    zi64 = jnp.zeros((_RPW, 64), jnp.int32)
    ids0 = _sc_extract(sp0, n0.reshape(_ROWS), zi128, 1024, 128)
    ids1 = _sc_extract(sp1, n1.reshape(_ROWS), zi128, 1024, 128)
    ids2 = _sc_extract(sp2, n2.reshape(_ROWS), zi64, 512, 64)

    return (ids0.reshape(_SEQ, _B, 128), ids1.reshape(_SEQ, _B, 128),
            ids2.reshape(_SEQ, _B, 64),
            n0.reshape(_SEQ, _B, 1), n1.reshape(_SEQ, _B, 1),
            n2.reshape(_SEQ, _B, 1),
            st0.reshape(_SEQ, _B, 1024), st1.reshape(_SEQ, _B, 1024),
            st2.reshape(_SEQ, _B, 512))


# speedup vs baseline: 1.3527x; 1.0365x over previous
import functools

import jax
import jax.numpy as jnp
from jax.experimental import pallas as pl
from jax.experimental.pallas import tpu as pltpu
from jax.experimental.pallas import tpu_sc as plsc

_SEQ = 512
_B = 8
_ROWS = _SEQ * _B
_NC = 2
_NS = 16
_NW = _NC * _NS
_RPW = _ROWS // _NW
_L = 16
_S_IN = 64


def _sc_mesh():
    return plsc.VectorSubcoreMesh(core_axis_name="c", subcore_axis_name="s")


def _worker_id():
    return jax.lax.axis_index("s") * _NC + jax.lax.axis_index("c")


def _sc_scatter(ids_flat, nums_flat, zeros2d):
    din = 512

    @functools.partial(
        pl.kernel,
        mesh=_sc_mesh(),
        compiler_params=pltpu.CompilerParams(
            needs_layout_passes=False, use_tc_tiling_on_sc=True),
        out_type=jax.ShapeDtypeStruct((_ROWS, din), jnp.float32),
        scratch_types=[
            pltpu.VMEM((_RPW * _S_IN,), jnp.int32),
            pltpu.VMEM((_RPW,), jnp.int32),
            pltpu.VMEM((_RPW, din), jnp.float32),
        ],
    )
    def k(ids_hbm, nums_hbm, z_hbm, out_hbm, ids_v, nums_v, buf_v):
        wid = _worker_id()
        base = wid * _RPW
        pltpu.sync_copy(ids_hbm.at[pl.ds(base * _S_IN, _RPW * _S_IN)], ids_v)
        pltpu.sync_copy(nums_hbm.at[pl.ds(base, _RPW)], nums_v)
        pltpu.sync_copy(z_hbm, buf_v)
        ones = jnp.full((_L,), 1.0, jnp.float32)
        lane = jax.lax.iota(jnp.int32, _L)

        def group(g, carry):
            numv = nums_v[pl.ds(g * _L, _L)]
            for j in range(_L):
                r = g * _L + j
                num = numv[j]
                rsp = jnp.full((_L,), 0, jnp.int32) + r
                for c in range(_S_IN // _L):
                    idv = ids_v[pl.ds(r * _S_IN + c * _L, _L)]
                    m = (lane + (c * _L)) < num
                    plsc.store_scatter(buf_v, [rsp, idv], ones, mask=m)
            return carry

        jax.lax.fori_loop(0, _RPW // _L, group, 0)
        pltpu.sync_copy(buf_v, out_hbm.at[pl.ds(base, _RPW)])

    return k(ids_flat, nums_flat, zeros2d)


def _sc_extract(sp2d, cnt_flat, zeros2d, d, kk):
    pad = kk + 2 * _L

    @functools.partial(
        pl.kernel,
        mesh=_sc_mesh(),
        compiler_params=pltpu.CompilerParams(
            needs_layout_passes=False, use_tc_tiling_on_sc=True),
        out_type=jax.ShapeDtypeStruct((_ROWS, kk), jnp.int32),
        scratch_types=[
            pltpu.VMEM((_RPW,), jnp.int32),
            pltpu.VMEM((_RPW, kk), jnp.int32),
            pltpu.VMEM((1, d), jnp.float32),
            pltpu.VMEM((pad,), jnp.int32),
        ],
    )
    def k(sp_hbm, cnt_hbm, z_hbm, out_hbm, cnt_v, buf_v, row_v, ext_v):
        wid = _worker_id()
        base = wid * _RPW
        pltpu.sync_copy(cnt_hbm.at[pl.ds(base, _RPW)], cnt_v)
        pltpu.sync_copy(z_hbm, buf_v)
        lane = jax.lax.iota(jnp.int32, _L)
        zv = jnp.zeros((_L,), jnp.int32)

        def group(g, carry):
            cntv = cnt_v[pl.ds(g * _L, _L)]
            for j in range(_L):
                r = g * _L + j
                cnt = cntv[j]

                @pl.when(cnt > 0)
                def _(r=r):
                    pltpu.sync_copy(sp_hbm.at[pl.ds(base + r, 1)], row_v)
                    for z in range(pad // _L):
                        ext_v[pl.ds(z * _L, _L)] = zv

                    def chunk(c, off):
                        v = row_v[0, pl.ds(c * _L, _L)]
                        m = v > 0.0
                        s = jnp.sum(m.astype(jnp.int32))

                        @pl.when((s > 0) & (off < kk))
                        def _():
                            plsc.store_compressed(
                                ext_v.at[pl.ds(off, _L)], lane + (c * _L),
                                mask=m)

                        return off + s

                    jax.lax.fori_loop(0, d // _L, chunk, 0)
                    for z in range(kk // _L):
                        buf_v[r, pl.ds(z * _L, _L)] = ext_v[pl.ds(z * _L, _L)]

            return carry

        jax.lax.fori_loop(0, _RPW // _L, group, 0)
        pltpu.sync_copy(buf_v, out_hbm.at[pl.ds(base, _RPW)])

    return k(sp2d, cnt_flat, zeros2d)


_CHUNK = 64
_RC = _CHUNK * _B


def _lif_chunk(syn_scr_ref, dec_ref, thr_ref, v_scr, st_ref, sp_ref, n_ref,
               kk):
    dec = dec_ref[...]
    om = 1.0 - dec
    thr = thr_ref[...]
    zero = jnp.zeros((), jnp.float32)
    one = jnp.ones((), jnp.float32)

    def step(i, v):
        rs = pl.ds(i * _B, _B)
        v = dec * v + om * syn_scr_ref[rs]
        fired = v > thr
        sp = jnp.where(fired, one, zero)
        v = jnp.where(fired, zero, v)
        st_ref[rs] = v
        sp_ref[rs] = sp
        cnt = jnp.sum(sp, axis=1, keepdims=True)
        n_ref[rs] = jnp.minimum(cnt, float(kk)).astype(jnp.int32)
        return v

    v_scr[...] = jax.lax.fori_loop(0, _CHUNK, step, v_scr[...], unroll=32)


def _mm(x, w_ref):
    return jax.lax.dot_general(
        x, w_ref[...], (((1,), (1,)), ((), ())),
        preferred_element_type=jnp.float32)


def _fused_body(dense_ref, w0_ref, w1_ref, w2_ref,
                dec0_ref, thr0_ref, init0_ref,
                dec1_ref, thr1_ref, init1_ref,
                dec2_ref, thr2_ref, init2_ref,
                st0_ref, sp0_ref, n0_ref,
                st1_ref, sp1_ref, n1_ref,
                st2_ref, sp2_ref, n2_ref,
                syn_scr, syn2_scr, v0_scr, v1_scr, v2_scr):
    @pl.when(pl.program_id(0) == 0)
    def _():
        v0_scr[...] = init0_ref[...]
        v1_scr[...] = init1_ref[...]
        v2_scr[...] = init2_ref[...]

    syn_scr[...] = _mm(dense_ref[...], w0_ref)
    _lif_chunk(syn_scr, dec0_ref, thr0_ref, v0_scr, st0_ref, sp0_ref,
               n0_ref, 128)
    syn_scr[...] = _mm(sp0_ref[...], w1_ref)
    _lif_chunk(syn_scr, dec1_ref, thr1_ref, v1_scr, st1_ref, sp1_ref,
               n1_ref, 128)
    syn2_scr[...] = _mm(sp1_ref[...], w2_ref)
    _lif_chunk(syn2_scr, dec2_ref, thr2_ref, v2_scr, st2_ref, sp2_ref,
               n2_ref, 64)


def _fused_net(dense, w0, w1, w2, dec0, thr0, init0, dec1, thr1, init1,
               dec2, thr2, init2):
    const = lambda i: (0, 0)
    rowblk = lambda i: (i, 0)
    return pl.pallas_call(
        _fused_body,
        grid=(_ROWS // _RC,),
        in_specs=[
            pl.BlockSpec((_RC, 512), rowblk),
            pl.BlockSpec((1024, 512), const),
            pl.BlockSpec((1024, 1024), const),
            pl.BlockSpec((512, 1024), const),
            pl.BlockSpec((1, 1024), const),
            pl.BlockSpec((1, 1024), const),
            pl.BlockSpec((_B, 1024), const),
            pl.BlockSpec((1, 1024), const),
            pl.BlockSpec((1, 1024), const),
            pl.BlockSpec((_B, 1024), const),
            pl.BlockSpec((1, 512), const),
            pl.BlockSpec((1, 512), const),
            pl.BlockSpec((_B, 512), const),
        ],
        out_specs=[
            pl.BlockSpec((_RC, 1024), rowblk),
            pl.BlockSpec((_RC, 1024), rowblk),
            pl.BlockSpec((_RC, 1), rowblk),
            pl.BlockSpec((_RC, 1024), rowblk),
            pl.BlockSpec((_RC, 1024), rowblk),
            pl.BlockSpec((_RC, 1), rowblk),
            pl.BlockSpec((_RC, 512), rowblk),
            pl.BlockSpec((_RC, 512), rowblk),
            pl.BlockSpec((_RC, 1), rowblk),
        ],
        out_shape=[
            jax.ShapeDtypeStruct((_ROWS, 1024), jnp.float32),
            jax.ShapeDtypeStruct((_ROWS, 1024), jnp.float32),
            jax.ShapeDtypeStruct((_ROWS, 1), jnp.int32),
            jax.ShapeDtypeStruct((_ROWS, 1024), jnp.float32),
            jax.ShapeDtypeStruct((_ROWS, 1024), jnp.float32),
            jax.ShapeDtypeStruct((_ROWS, 1), jnp.int32),
            jax.ShapeDtypeStruct((_ROWS, 512), jnp.float32),
            jax.ShapeDtypeStruct((_ROWS, 512), jnp.float32),
            jax.ShapeDtypeStruct((_ROWS, 1), jnp.int32),
        ],
        scratch_shapes=[
            pltpu.VMEM((_RC, 1024), jnp.float32),
            pltpu.VMEM((_RC, 512), jnp.float32),
            pltpu.VMEM((_B, 1024), jnp.float32),
            pltpu.VMEM((_B, 1024), jnp.float32),
            pltpu.VMEM((_B, 512), jnp.float32),
        ],
    )(dense, w0, w1, w2, dec0.reshape(1, 1024), thr0.reshape(1, 1024), init0,
      dec1.reshape(1, 1024), thr1.reshape(1, 1024), init1,
      dec2.reshape(1, 512), thr2.reshape(1, 512), init2)


def kernel(inp_spike_ids, num_inp_spikes, init_state_0, init_state_1,
           init_state_2, w0, w1, w2, decay_0, decay_1, decay_2,
           thr_0, thr_1, thr_2):
    ids_flat = inp_spike_ids.reshape(_ROWS * _S_IN)
    nums_flat = num_inp_spikes.reshape(_ROWS)
    zf = jnp.zeros((_RPW, 512), jnp.float32)

    dense = _sc_scatter(ids_flat, nums_flat, zf)

    st0, sp0, n0, st1, sp1, n1, st2, sp2, n2 = _fused_net(
        dense, w0, w1, w2, decay_0, thr_0, init_state_0,
        decay_1, thr_1, init_state_1, decay_2, thr_2, init_state_2)

    zi128 = jnp.zeros((_RPW, 128), jnp.int32)
    zi64 = jnp.zeros((_RPW, 64), jnp.int32)
    ids0 = _sc_extract(sp0, n0.reshape(_ROWS), zi128, 1024, 128)
    ids1 = _sc_extract(sp1, n1.reshape(_ROWS), zi128, 1024, 128)
    ids2 = _sc_extract(sp2, n2.reshape(_ROWS), zi64, 512, 64)

    return (ids0.reshape(_SEQ, _B, 128), ids1.reshape(_SEQ, _B, 128),
            ids2.reshape(_SEQ, _B, 64),
            n0.reshape(_SEQ, _B, 1), n1.reshape(_SEQ, _B, 1),
            n2.reshape(_SEQ, _B, 1),
            st0.reshape(_SEQ, _B, 1024), st1.reshape(_SEQ, _B, 1024),
            st2.reshape(_SEQ, _B, 512))


# speedup vs baseline: 1.4911x; 1.1023x over previous
import functools

import jax
import jax.numpy as jnp
from jax.experimental import pallas as pl
from jax.experimental.pallas import tpu as pltpu
from jax.experimental.pallas import tpu_sc as plsc

_SEQ = 512
_B = 8
_ROWS = _SEQ * _B
_NC = 2
_NS = 16
_NW = _NC * _NS
_RPW = _ROWS // _NW
_L = 16
_S_IN = 64


def _sc_mesh():
    return plsc.VectorSubcoreMesh(core_axis_name="c", subcore_axis_name="s")


def _worker_id():
    return jax.lax.axis_index("s") * _NC + jax.lax.axis_index("c")


def _sc_scatter(ids_flat, nums_flat, zeros2d):
    din = 512

    @functools.partial(
        pl.kernel,
        mesh=_sc_mesh(),
        compiler_params=pltpu.CompilerParams(
            needs_layout_passes=False, use_tc_tiling_on_sc=True),
        out_type=jax.ShapeDtypeStruct((_ROWS, din), jnp.float32),
        scratch_types=[
            pltpu.VMEM((_RPW * _S_IN,), jnp.int32),
            pltpu.VMEM((_RPW,), jnp.int32),
            pltpu.VMEM((_RPW, din), jnp.float32),
        ],
    )
    def k(ids_hbm, nums_hbm, z_hbm, out_hbm, ids_v, nums_v, buf_v):
        wid = _worker_id()
        base = wid * _RPW
        pltpu.sync_copy(ids_hbm.at[pl.ds(base * _S_IN, _RPW * _S_IN)], ids_v)
        pltpu.sync_copy(nums_hbm.at[pl.ds(base, _RPW)], nums_v)
        pltpu.sync_copy(z_hbm, buf_v)
        ones = jnp.full((_L,), 1.0, jnp.float32)
        lane = jax.lax.iota(jnp.int32, _L)

        def group(g, carry):
            numv = nums_v[pl.ds(g * _L, _L)]
            for j in range(_L):
                r = g * _L + j
                num = numv[j]
                rsp = jnp.full((_L,), 0, jnp.int32) + r
                for c in range(_S_IN // _L):
                    idv = ids_v[pl.ds(r * _S_IN + c * _L, _L)]
                    m = (lane + (c * _L)) < num
                    plsc.store_scatter(buf_v, [rsp, idv], ones, mask=m)
            return carry

        jax.lax.fori_loop(0, _RPW // _L, group, 0)
        pltpu.sync_copy(buf_v, out_hbm.at[pl.ds(base, _RPW)])

    return k(ids_flat, nums_flat, zeros2d)


def _sc_extract(sp2d, cnt_flat, zeros2d, d, kk):
    pad = kk + 2 * _L

    @functools.partial(
        pl.kernel,
        mesh=_sc_mesh(),
        compiler_params=pltpu.CompilerParams(
            needs_layout_passes=False, use_tc_tiling_on_sc=True),
        out_type=jax.ShapeDtypeStruct((_ROWS, kk), jnp.int32),
        scratch_types=[
            pltpu.VMEM((_RPW,), jnp.int32),
            pltpu.VMEM((_RPW, kk), jnp.int32),
            pltpu.VMEM((1, d), jnp.float32),
            pltpu.VMEM((pad,), jnp.int32),
        ],
    )
    def k(sp_hbm, cnt_hbm, z_hbm, out_hbm, cnt_v, buf_v, row_v, ext_v):
        wid = _worker_id()
        base = wid * _RPW
        pltpu.sync_copy(cnt_hbm.at[pl.ds(base, _RPW)], cnt_v)
        pltpu.sync_copy(z_hbm, buf_v)
        lane = jax.lax.iota(jnp.int32, _L)
        zv = jnp.zeros((_L,), jnp.int32)

        def group(g, carry):
            cntv = cnt_v[pl.ds(g * _L, _L)]
            for j in range(_L):
                r = g * _L + j
                cnt = cntv[j]

                @pl.when(cnt > 0)
                def _(r=r):
                    pltpu.sync_copy(sp_hbm.at[pl.ds(base + r, 1)], row_v)
                    for z in range(pad // _L):
                        ext_v[pl.ds(z * _L, _L)] = zv

                    def chunk(c, off):
                        v = row_v[0, pl.ds(c * _L, _L)]
                        m = v > 0.0
                        s = jnp.sum(m.astype(jnp.int32))

                        @pl.when((s > 0) & (off < kk))
                        def _():
                            plsc.store_compressed(
                                ext_v.at[pl.ds(off, _L)], lane + (c * _L),
                                mask=m)

                        return off + s

                    jax.lax.fori_loop(0, d // _L, chunk, 0)
                    for z in range(kk // _L):
                        buf_v[r, pl.ds(z * _L, _L)] = ext_v[pl.ds(z * _L, _L)]

            return carry

        jax.lax.fori_loop(0, _RPW // _L, group, 0)
        pltpu.sync_copy(buf_v, out_hbm.at[pl.ds(base, _RPW)])

    return k(sp2d, cnt_flat, zeros2d)


_CHUNK = 64
_RC = _CHUNK * _B


def _lif_chunk(syn_scr_ref, dec_ref, thr_ref, v_scr, st_ref, sp_ref, n_ref,
               kk):
    dec = dec_ref[...]
    om = 1.0 - dec
    thr = thr_ref[...]
    zero = jnp.zeros((), jnp.float32)
    one = jnp.ones((), jnp.float32)

    def step(i, v):
        rs = pl.ds(i * _B, _B)
        v = dec * v + om * syn_scr_ref[rs]
        fired = v > thr
        sp = jnp.where(fired, one, zero)
        v = jnp.where(fired, zero, v)
        st_ref[rs] = v
        sp_ref[rs] = sp
        cnt = jnp.sum(sp, axis=1, keepdims=True)
        n_ref[rs] = jnp.minimum(cnt, float(kk)).astype(jnp.int32)
        return v

    v_scr[...] = jax.lax.fori_loop(0, _CHUNK, step, v_scr[...], unroll=64)


def _mm(x, w_ref):
    return jax.lax.dot_general(
        x, w_ref[...], (((1,), (1,)), ((), ())),
        preferred_element_type=jnp.float32)


def _fused_body(dense_ref, w0_ref, w1_ref, w2_ref,
                dec0_ref, thr0_ref, init0_ref,
                dec1_ref, thr1_ref, init1_ref,
                dec2_ref, thr2_ref, init2_ref,
                st0_ref, sp0_ref, n0_ref,
                st1_ref, sp1_ref, n1_ref,
                st2_ref, sp2_ref, n2_ref,
                syn_scr, syn2_scr, v0_scr, v1_scr, v2_scr):
    @pl.when(pl.program_id(0) == 0)
    def _():
        v0_scr[...] = init0_ref[...]
        v1_scr[...] = init1_ref[...]
        v2_scr[...] = init2_ref[...]

    syn_scr[...] = _mm(dense_ref[...], w0_ref)
    _lif_chunk(syn_scr, dec0_ref, thr0_ref, v0_scr, st0_ref, sp0_ref,
               n0_ref, 128)
    syn_scr[...] = _mm(sp0_ref[...], w1_ref)
    _lif_chunk(syn_scr, dec1_ref, thr1_ref, v1_scr, st1_ref, sp1_ref,
               n1_ref, 128)
    syn2_scr[...] = _mm(sp1_ref[...], w2_ref)
    _lif_chunk(syn2_scr, dec2_ref, thr2_ref, v2_scr, st2_ref, sp2_ref,
               n2_ref, 64)


def _fused_net(dense, w0, w1, w2, dec0, thr0, init0, dec1, thr1, init1,
               dec2, thr2, init2):
    const = lambda i: (0, 0)
    rowblk = lambda i: (i, 0)
    return pl.pallas_call(
        _fused_body,
        grid=(_ROWS // _RC,),
        in_specs=[
            pl.BlockSpec((_RC, 512), rowblk),
            pl.BlockSpec((1024, 512), const),
            pl.BlockSpec((1024, 1024), const),
            pl.BlockSpec((512, 1024), const),
            pl.BlockSpec((1, 1024), const),
            pl.BlockSpec((1, 1024), const),
            pl.BlockSpec((_B, 1024), const),
            pl.BlockSpec((1, 1024), const),
            pl.BlockSpec((1, 1024), const),
            pl.BlockSpec((_B, 1024), const),
            pl.BlockSpec((1, 512), const),
            pl.BlockSpec((1, 512), const),
            pl.BlockSpec((_B, 512), const),
        ],
        out_specs=[
            pl.BlockSpec((_RC, 1024), rowblk),
            pl.BlockSpec((_RC, 1024), rowblk),
            pl.BlockSpec((_RC, 1), rowblk),
            pl.BlockSpec((_RC, 1024), rowblk),
            pl.BlockSpec((_RC, 1024), rowblk),
            pl.BlockSpec((_RC, 1), rowblk),
            pl.BlockSpec((_RC, 512), rowblk),
            pl.BlockSpec((_RC, 512), rowblk),
            pl.BlockSpec((_RC, 1), rowblk),
        ],
        out_shape=[
            jax.ShapeDtypeStruct((_ROWS, 1024), jnp.float32),
            jax.ShapeDtypeStruct((_ROWS, 1024), jnp.float32),
            jax.ShapeDtypeStruct((_ROWS, 1), jnp.int32),
            jax.ShapeDtypeStruct((_ROWS, 1024), jnp.float32),
            jax.ShapeDtypeStruct((_ROWS, 1024), jnp.float32),
            jax.ShapeDtypeStruct((_ROWS, 1), jnp.int32),
            jax.ShapeDtypeStruct((_ROWS, 512), jnp.float32),
            jax.ShapeDtypeStruct((_ROWS, 512), jnp.float32),
            jax.ShapeDtypeStruct((_ROWS, 1), jnp.int32),
        ],
        scratch_shapes=[
            pltpu.VMEM((_RC, 1024), jnp.float32),
            pltpu.VMEM((_RC, 512), jnp.float32),
            pltpu.VMEM((_B, 1024), jnp.float32),
            pltpu.VMEM((_B, 1024), jnp.float32),
            pltpu.VMEM((_B, 512), jnp.float32),
        ],
    )(dense, w0, w1, w2, dec0.reshape(1, 1024), thr0.reshape(1, 1024), init0,
      dec1.reshape(1, 1024), thr1.reshape(1, 1024), init1,
      dec2.reshape(1, 512), thr2.reshape(1, 512), init2)


def kernel(inp_spike_ids, num_inp_spikes, init_state_0, init_state_1,
           init_state_2, w0, w1, w2, decay_0, decay_1, decay_2,
           thr_0, thr_1, thr_2):
    ids_flat = inp_spike_ids.reshape(_ROWS * _S_IN)
    nums_flat = num_inp_spikes.reshape(_ROWS)
    zf = jnp.zeros((_RPW, 512), jnp.float32)

    dense = _sc_scatter(ids_flat, nums_flat, zf)

    st0, sp0, n0, st1, sp1, n1, st2, sp2, n2 = _fused_net(
        dense, w0, w1, w2, decay_0, thr_0, init_state_0,
        decay_1, thr_1, init_state_1, decay_2, thr_2, init_state_2)

    zi128 = jnp.zeros((_RPW, 128), jnp.int32)
    zi64 = jnp.zeros((_RPW, 64), jnp.int32)
    ids0 = _sc_extract(sp0, n0.reshape(_ROWS), zi128, 1024, 128)
    ids1 = _sc_extract(sp1, n1.reshape(_ROWS), zi128, 1024, 128)
    ids2 = _sc_extract(sp2, n2.reshape(_ROWS), zi64, 512, 64)

    return (ids0.reshape(_SEQ, _B, 128), ids1.reshape(_SEQ, _B, 128),
            ids2.reshape(_SEQ, _B, 64),
            n0.reshape(_SEQ, _B, 1), n1.reshape(_SEQ, _B, 1),
            n2.reshape(_SEQ, _B, 1),
            st0.reshape(_SEQ, _B, 1024), st1.reshape(_SEQ, _B, 1024),
            st2.reshape(_SEQ, _B, 512))


# speedup vs baseline: 1.5687x; 1.0520x over previous
import functools

import jax
import jax.numpy as jnp
from jax.experimental import pallas as pl
from jax.experimental.pallas import tpu as pltpu
from jax.experimental.pallas import tpu_sc as plsc

_SEQ = 512
_B = 8
_ROWS = _SEQ * _B
_NC = 2
_NS = 16
_NW = _NC * _NS
_RPW = _ROWS // _NW
_L = 16
_S_IN = 64


def _sc_mesh():
    return plsc.VectorSubcoreMesh(core_axis_name="c", subcore_axis_name="s")


def _worker_id():
    return jax.lax.axis_index("s") * _NC + jax.lax.axis_index("c")


def _sc_scatter(ids_flat, nums_flat, zeros2d):
    din = 512

    @functools.partial(
        pl.kernel,
        mesh=_sc_mesh(),
        compiler_params=pltpu.CompilerParams(
            needs_layout_passes=False, use_tc_tiling_on_sc=True),
        out_type=jax.ShapeDtypeStruct((_ROWS, din), jnp.float32),
        scratch_types=[
            pltpu.VMEM((_RPW * _S_IN,), jnp.int32),
            pltpu.VMEM((_RPW,), jnp.int32),
            pltpu.VMEM((_RPW, din), jnp.float32),
        ],
    )
    def k(ids_hbm, nums_hbm, z_hbm, out_hbm, ids_v, nums_v, buf_v):
        wid = _worker_id()
        base = wid * _RPW
        pltpu.sync_copy(ids_hbm.at[pl.ds(base * _S_IN, _RPW * _S_IN)], ids_v)
        pltpu.sync_copy(nums_hbm.at[pl.ds(base, _RPW)], nums_v)
        pltpu.sync_copy(z_hbm, buf_v)
        ones = jnp.full((_L,), 1.0, jnp.float32)
        lane = jax.lax.iota(jnp.int32, _L)

        def group(g, carry):
            numv = nums_v[pl.ds(g * _L, _L)]
            for j in range(_L):
                r = g * _L + j
                num = numv[j]
                rsp = jnp.full((_L,), 0, jnp.int32) + r
                for c in range(_S_IN // _L):
                    idv = ids_v[pl.ds(r * _S_IN + c * _L, _L)]
                    m = (lane + (c * _L)) < num
                    plsc.store_scatter(buf_v, [rsp, idv], ones, mask=m)
            return carry

        jax.lax.fori_loop(0, _RPW // _L, group, 0)
        pltpu.sync_copy(buf_v, out_hbm.at[pl.ds(base, _RPW)])

    return k(ids_flat, nums_flat, zeros2d)


def _sc_extract3(sp0, cnt0, sp1, cnt1, sp2, cnt2, z128, z64):
    pad = 128 + 2 * _L

    @functools.partial(
        pl.kernel,
        mesh=_sc_mesh(),
        compiler_params=pltpu.CompilerParams(
            needs_layout_passes=False, use_tc_tiling_on_sc=True),
        out_type=[
            jax.ShapeDtypeStruct((_ROWS, 128), jnp.int32),
            jax.ShapeDtypeStruct((_ROWS, 128), jnp.int32),
            jax.ShapeDtypeStruct((_ROWS, 64), jnp.int32),
        ],
        scratch_types=[
            pltpu.VMEM((_RPW,), jnp.int32),
            pltpu.VMEM((_RPW,), jnp.int32),
            pltpu.VMEM((_RPW,), jnp.int32),
            pltpu.VMEM((_RPW, 128), jnp.int32),
            pltpu.VMEM((_RPW, 128), jnp.int32),
            pltpu.VMEM((_RPW, 64), jnp.int32),
            pltpu.VMEM((1, 1024), jnp.float32),
            pltpu.VMEM((1, 512), jnp.float32),
            pltpu.VMEM((pad,), jnp.int32),
        ],
    )
    def k(sp0_hbm, cnt0_hbm, sp1_hbm, cnt1_hbm, sp2_hbm, cnt2_hbm,
          z128_hbm, z64_hbm, out0_hbm, out1_hbm, out2_hbm,
          cnt0_v, cnt1_v, cnt2_v, buf0_v, buf1_v, buf2_v,
          row_v, row2_v, ext_v):
        wid = _worker_id()
        base = wid * _RPW
        pltpu.sync_copy(cnt0_hbm.at[pl.ds(base, _RPW)], cnt0_v)
        pltpu.sync_copy(cnt1_hbm.at[pl.ds(base, _RPW)], cnt1_v)
        pltpu.sync_copy(cnt2_hbm.at[pl.ds(base, _RPW)], cnt2_v)
        pltpu.sync_copy(z128_hbm, buf0_v)
        pltpu.sync_copy(z128_hbm, buf1_v)
        pltpu.sync_copy(z64_hbm, buf2_v)
        lane = jax.lax.iota(jnp.int32, _L)
        zv = jnp.zeros((_L,), jnp.int32)

        def layer(sp_hbm, cnt_v, buf_v, rv, d, kk):
            def group(g, carry):
                cntv = cnt_v[pl.ds(g * _L, _L)]
                for j in range(_L):
                    r = g * _L + j
                    cnt = cntv[j]

                    @pl.when(cnt > 0)
                    def _(r=r):
                        pltpu.sync_copy(sp_hbm.at[pl.ds(base + r, 1)], rv)
                        for z in range((kk + 2 * _L) // _L):
                            ext_v[pl.ds(z * _L, _L)] = zv

                        def chunk(c, off):
                            v = rv[0, pl.ds(c * _L, _L)]
                            m = v > 0.0
                            s = jnp.sum(m.astype(jnp.int32))

                            @pl.when((s > 0) & (off < kk))
                            def _():
                                plsc.store_compressed(
                                    ext_v.at[pl.ds(off, _L)],
                                    lane + (c * _L), mask=m)

                            return off + s

                        jax.lax.fori_loop(0, d // _L, chunk, 0)
                        for z in range(kk // _L):
                            buf_v[r, pl.ds(z * _L, _L)] = \
                                ext_v[pl.ds(z * _L, _L)]

                return carry

            jax.lax.fori_loop(0, _RPW // _L, group, 0)

        layer(sp0_hbm, cnt0_v, buf0_v, row_v, 1024, 128)
        layer(sp1_hbm, cnt1_v, buf1_v, row_v, 1024, 128)
        layer(sp2_hbm, cnt2_v, buf2_v, row2_v, 512, 64)
        pltpu.sync_copy(buf0_v, out0_hbm.at[pl.ds(base, _RPW)])
        pltpu.sync_copy(buf1_v, out1_hbm.at[pl.ds(base, _RPW)])
        pltpu.sync_copy(buf2_v, out2_hbm.at[pl.ds(base, _RPW)])

    return k(sp0, cnt0, sp1, cnt1, sp2, cnt2, z128, z64)


_CHUNK = 64
_RC = _CHUNK * _B


def _lif_chunk(syn_scr_ref, dec_ref, thr_ref, v_scr, st_ref, sp_ref, n_ref,
               kk):
    dec = dec_ref[...]
    om = 1.0 - dec
    thr = thr_ref[...]
    zero = jnp.zeros((), jnp.float32)
    one = jnp.ones((), jnp.float32)

    def step(i, v):
        rs = pl.ds(i * _B, _B)
        v = dec * v + om * syn_scr_ref[rs]
        fired = v > thr
        sp = jnp.where(fired, one, zero)
        v = jnp.where(fired, zero, v)
        st_ref[rs] = v
        sp_ref[rs] = sp
        cnt = jnp.sum(sp, axis=1, keepdims=True)
        n_ref[rs] = jnp.minimum(cnt, float(kk)).astype(jnp.int32)
        return v

    v_scr[...] = jax.lax.fori_loop(0, _CHUNK, step, v_scr[...], unroll=64)


def _mm(x, w_ref):
    return jax.lax.dot_general(
        x, w_ref[...], (((1,), (1,)), ((), ())),
        preferred_element_type=jnp.float32)


def _fused_body(dense_ref, w0_ref, w1_ref, w2_ref,
                dec0_ref, thr0_ref, init0_ref,
                dec1_ref, thr1_ref, init1_ref,
                dec2_ref, thr2_ref, init2_ref,
                st0_ref, sp0_ref, n0_ref,
                st1_ref, sp1_ref, n1_ref,
                st2_ref, sp2_ref, n2_ref,
                syn_scr, syn2_scr, v0_scr, v1_scr, v2_scr):
    @pl.when(pl.program_id(0) == 0)
    def _():
        v0_scr[...] = init0_ref[...]
        v1_scr[...] = init1_ref[...]
        v2_scr[...] = init2_ref[...]

    syn_scr[...] = _mm(dense_ref[...], w0_ref)
    _lif_chunk(syn_scr, dec0_ref, thr0_ref, v0_scr, st0_ref, sp0_ref,
               n0_ref, 128)
    syn_scr[...] = _mm(sp0_ref[...], w1_ref)
    _lif_chunk(syn_scr, dec1_ref, thr1_ref, v1_scr, st1_ref, sp1_ref,
               n1_ref, 128)
    syn2_scr[...] = _mm(sp1_ref[...], w2_ref)
    _lif_chunk(syn2_scr, dec2_ref, thr2_ref, v2_scr, st2_ref, sp2_ref,
               n2_ref, 64)


def _fused_net(dense, w0, w1, w2, dec0, thr0, init0, dec1, thr1, init1,
               dec2, thr2, init2):
    const = lambda i: (0, 0)
    rowblk = lambda i: (i, 0)
    return pl.pallas_call(
        _fused_body,
        grid=(_ROWS // _RC,),
        in_specs=[
            pl.BlockSpec((_RC, 512), rowblk),
            pl.BlockSpec((1024, 512), const),
            pl.BlockSpec((1024, 1024), const),
            pl.BlockSpec((512, 1024), const),
            pl.BlockSpec((1, 1024), const),
            pl.BlockSpec((1, 1024), const),
            pl.BlockSpec((_B, 1024), const),
            pl.BlockSpec((1, 1024), const),
            pl.BlockSpec((1, 1024), const),
            pl.BlockSpec((_B, 1024), const),
            pl.BlockSpec((1, 512), const),
            pl.BlockSpec((1, 512), const),
            pl.BlockSpec((_B, 512), const),
        ],
        out_specs=[
            pl.BlockSpec((_RC, 1024), rowblk),
            pl.BlockSpec((_RC, 1024), rowblk),
            pl.BlockSpec((_RC, 1), rowblk),
            pl.BlockSpec((_RC, 1024), rowblk),
            pl.BlockSpec((_RC, 1024), rowblk),
            pl.BlockSpec((_RC, 1), rowblk),
            pl.BlockSpec((_RC, 512), rowblk),
            pl.BlockSpec((_RC, 512), rowblk),
            pl.BlockSpec((_RC, 1), rowblk),
        ],
        out_shape=[
            jax.ShapeDtypeStruct((_ROWS, 1024), jnp.float32),
            jax.ShapeDtypeStruct((_ROWS, 1024), jnp.float32),
            jax.ShapeDtypeStruct((_ROWS, 1), jnp.int32),
            jax.ShapeDtypeStruct((_ROWS, 1024), jnp.float32),
            jax.ShapeDtypeStruct((_ROWS, 1024), jnp.float32),
            jax.ShapeDtypeStruct((_ROWS, 1), jnp.int32),
            jax.ShapeDtypeStruct((_ROWS, 512), jnp.float32),
            jax.ShapeDtypeStruct((_ROWS, 512), jnp.float32),
            jax.ShapeDtypeStruct((_ROWS, 1), jnp.int32),
        ],
        scratch_shapes=[
            pltpu.VMEM((_RC, 1024), jnp.float32),
            pltpu.VMEM((_RC, 512), jnp.float32),
            pltpu.VMEM((_B, 1024), jnp.float32),
            pltpu.VMEM((_B, 1024), jnp.float32),
            pltpu.VMEM((_B, 512), jnp.float32),
        ],
    )(dense, w0, w1, w2, dec0.reshape(1, 1024), thr0.reshape(1, 1024), init0,
      dec1.reshape(1, 1024), thr1.reshape(1, 1024), init1,
      dec2.reshape(1, 512), thr2.reshape(1, 512), init2)


def kernel(inp_spike_ids, num_inp_spikes, init_state_0, init_state_1,
           init_state_2, w0, w1, w2, decay_0, decay_1, decay_2,
           thr_0, thr_1, thr_2):
    ids_flat = inp_spike_ids.reshape(_ROWS * _S_IN)
    nums_flat = num_inp_spikes.reshape(_ROWS)
    zf = jnp.zeros((_RPW, 512), jnp.float32)

    dense = _sc_scatter(ids_flat, nums_flat, zf)

    st0, sp0, n0, st1, sp1, n1, st2, sp2, n2 = _fused_net(
        dense, w0, w1, w2, decay_0, thr_0, init_state_0,
        decay_1, thr_1, init_state_1, decay_2, thr_2, init_state_2)

    zi128 = jnp.zeros((_RPW, 128), jnp.int32)
    zi64 = jnp.zeros((_RPW, 64), jnp.int32)
    ids0, ids1, ids2 = _sc_extract3(
        sp0, n0.reshape(_ROWS), sp1, n1.reshape(_ROWS),
        sp2, n2.reshape(_ROWS), zi128, zi64)

    return (ids0.reshape(_SEQ, _B, 128), ids1.reshape(_SEQ, _B, 128),
            ids2.reshape(_SEQ, _B, 64),
            n0.reshape(_SEQ, _B, 1), n1.reshape(_SEQ, _B, 1),
            n2.reshape(_SEQ, _B, 1),
            st0.reshape(_SEQ, _B, 1024), st1.reshape(_SEQ, _B, 1024),
            st2.reshape(_SEQ, _B, 512))
